# Initial kernel scaffold; baseline (speedup 1.0000x reference)
import jax
import jax.numpy as jnp
from jax.experimental import pallas as pl


def kernel(X, table):
    raise NotImplementedError("write your pallas kernel here")


# speedup vs baseline: 1.3414x; 1.3414x over previous
import functools

import jax
import jax.numpy as jnp
from jax import lax
from jax.experimental import pallas as pl
from jax.experimental.pallas import tpu as pltpu
from jax.experimental.pallas import tpu_sc as plsc

VOCAB = 1000000
DIM = 32

_info = plsc.get_sparse_core_info()
_NC, _NS = _info.num_cores, _info.num_subcores
_NW = _NC * _NS

_IDXW = 128
_NB = 8


def _make_gather(n_rows: int):
  rows_per_w = n_rows // _NW
  n_chunks = rows_per_w // _NB
  mesh = plsc.VectorSubcoreMesh(core_axis_name="c", subcore_axis_name="s")

  @functools.partial(
      pl.kernel,
      mesh=mesh,
      compiler_params=pltpu.CompilerParams(use_tc_tiling_on_sc=False),
      out_type=jax.ShapeDtypeStruct((n_rows, _IDXW, DIM), jnp.float32),
      scratch_types=[
          pltpu.VMEM((_NB, _IDXW), jnp.int32),
          pltpu.VMEM((_NB, _IDXW, DIM), jnp.float32),
          pltpu.SemaphoreType.DMA,
      ],
  )
  def gather_kernel(idx_hbm, table_hbm, out_hbm, idx_v, rows_v, sem):
    wid = lax.axis_index("s") * _NC + lax.axis_index("c")
    base = wid * rows_per_w

    def chunk(i, carry):
      row0 = base + i * _NB
      pltpu.sync_copy(idx_hbm.at[pl.ds(row0, _NB)], idx_v)
      copies = []
      for j in range(_NB):
        copies.append(
            pltpu.async_copy(table_hbm.at[idx_v.at[j]], rows_v.at[j], sem))
      for c in copies:
        c.wait()
      pltpu.sync_copy(rows_v, out_hbm.at[pl.ds(row0, _NB)])
      return carry

    lax.fori_loop(0, n_chunks, chunk, 0)

  return gather_kernel


def kernel(X, table):
  n_flat = X.shape[0] * X.shape[1]
  idx2d = X.reshape(n_flat // _IDXW, _IDXW)
  out = _make_gather(idx2d.shape[0])(idx2d, table)
  return out.reshape(X.shape[0], X.shape[1], DIM)


# speedup vs baseline: 1.3655x; 1.0180x over previous
import functools

import jax
import jax.numpy as jnp
from jax import lax
from jax.experimental import pallas as pl
from jax.experimental.pallas import tpu as pltpu
from jax.experimental.pallas import tpu_sc as plsc

VOCAB = 1000000
DIM = 32

_info = plsc.get_sparse_core_info()
_NC, _NS = _info.num_cores, _info.num_subcores
_NW = _NC * _NS

_IDXW = 128
_NB = 10


def _make_gather(n_rows: int):
  rows_per_w = n_rows // _NW
  n_pairs = rows_per_w // (2 * _NB)
  mesh = plsc.VectorSubcoreMesh(core_axis_name="c", subcore_axis_name="s")

  @functools.partial(
      pl.kernel,
      mesh=mesh,
      compiler_params=pltpu.CompilerParams(use_tc_tiling_on_sc=False),
      out_type=jax.ShapeDtypeStruct((n_rows, _IDXW, DIM), jnp.float32),
      scratch_types=[
          pltpu.VMEM((2, _NB, _IDXW), jnp.int32),
          pltpu.VMEM((2, _NB, _IDXW, DIM), jnp.float32),
          pltpu.SemaphoreType.DMA,
          pltpu.SemaphoreType.DMA,
          pltpu.SemaphoreType.DMA,
          pltpu.SemaphoreType.DMA,
      ],
  )
  def gather_kernel(idx_hbm, table_hbm, out_hbm, idx_v, rows_v, g0, g1, o0,
                    o1):
    wid = lax.axis_index("s") * _NC + lax.axis_index("c")
    base = wid * rows_per_w
    osems = (o0, o1)

    def fire(row0, buf, gsem):
      pltpu.sync_copy(idx_hbm.at[pl.ds(row0, _NB)], idx_v.at[buf])
      return [
          pltpu.async_copy(table_hbm.at[idx_v.at[buf, j]],
                           rows_v.at[buf, j], gsem)
          for j in range(_NB)
      ]

    def pair(g, carry):
      a = base + g * 2 * _NB
      b = a + _NB

      @pl.when(g > 0)
      def _drain_prev():
        pltpu.make_async_copy(
            rows_v.at[0], out_hbm.at[pl.ds(a - 2 * _NB, _NB)], o0).wait()
        pltpu.make_async_copy(
            rows_v.at[1], out_hbm.at[pl.ds(a - _NB, _NB)], o1).wait()

      ca = fire(a, 0, g0)
      cb = fire(b, 1, g1)

      for c in ca:
        c.wait()
      pltpu.async_copy(rows_v.at[0], out_hbm.at[pl.ds(a, _NB)], o0)
      for c in cb:
        c.wait()
      pltpu.async_copy(rows_v.at[1], out_hbm.at[pl.ds(b, _NB)], o1)
      return carry

    lax.fori_loop(0, n_pairs, pair, 0)
    last_a = base + (n_pairs - 1) * 2 * _NB
    pltpu.make_async_copy(
        rows_v.at[0], out_hbm.at[pl.ds(last_a, _NB)], o0).wait()
    pltpu.make_async_copy(
        rows_v.at[1], out_hbm.at[pl.ds(last_a + _NB, _NB)], o1).wait()

  return gather_kernel


def kernel(X, table):
  n_flat = X.shape[0] * X.shape[1]
  idx2d = X.reshape(n_flat // _IDXW, _IDXW)
  out = _make_gather(idx2d.shape[0])(idx2d, table)
  return out.reshape(X.shape[0], X.shape[1], DIM)
